# baseline (speedup 1.0000x reference)
import functools

import jax
import jax.numpy as jnp
from jax import lax
from jax.experimental import pallas as pl
from jax.experimental.pallas import tpu as pltpu
from jax.experimental.pallas import tpu_sc as plsc


def _vq_body(x_ref, m_ref, c_ref, idx_ref, sq_ref):
    x = x_ref[0]
    c = c_ref[...]
    n = c.shape[0]
    xn = jnp.sum(x * x, axis=1, keepdims=True)
    cn = jnp.sum(c * c, axis=1)
    xc = lax.dot_general(x, c, (((1,), (1,)), ((), ())),
                         preferred_element_type=jnp.float32,
                         precision=lax.Precision.DEFAULT)
    dist = xn - 2.0 * xc + cn[None, :]
    mind = jnp.min(dist, axis=1, keepdims=True)
    ids = lax.broadcasted_iota(jnp.int32, dist.shape, 1)
    idx = jnp.min(jnp.where(dist == mind, ids, n), axis=1)
    idx_ref[0, 0] = idx
    sq_ref[0, 0] = mind[:, 0] * m_ref[0, 0]


_SC_INFO = plsc.get_sparse_core_info()
_NC = _SC_INFO.num_cores
_NS = _SC_INFO.num_subcores
_NW = _NC * _NS


@functools.lru_cache(maxsize=None)
def _make_sc_gather(V, D, Btot, CH):
    b_per_w = Btot // _NW
    nch = b_per_w // CH
    mesh = plsc.VectorSubcoreMesh(core_axis_name="c", subcore_axis_name="s")

    @functools.partial(
        pl.kernel, mesh=mesh,
        out_type=jax.ShapeDtypeStruct((Btot, D), jnp.float32),
        scratch_types=[
            pltpu.VMEM((b_per_w,), jnp.int32),
            pltpu.VMEM((2, CH, D), jnp.float32),
            pltpu.SemaphoreType.DMA,
            pltpu.SemaphoreType.DMA,
        ],
    )
    def gather_k(table_hbm, idx_hbm, out_hbm, idx_v, rows_v, gsem, ssem):
        wid = lax.axis_index("s") * _NC + lax.axis_index("c")
        base = wid * b_per_w
        pltpu.sync_copy(idx_hbm.at[pl.ds(base, b_per_w)], idx_v)

        def gather_start(i, slot):
            return pltpu.async_copy(
                table_hbm.at[idx_v.at[pl.ds(i * CH, CH)]],
                rows_v.at[slot], gsem)

        def write_start(i, slot):
            return pltpu.async_copy(
                rows_v.at[slot], out_hbm.at[pl.ds(base + i * CH, CH)], ssem)

        gather_start(0, 0).wait()
        for i in range(1, nch):
            g = gather_start(i, i % 2)
            write_start(i - 1, (i - 1) % 2)
            g.wait()
            pltpu.make_async_copy(
                rows_v.at[(i - 1) % 2],
                out_hbm.at[pl.ds(base + (i - 1) * CH, CH)], ssem).wait()
        write_start(nch - 1, (nch - 1) % 2).wait()

    return gather_k


def kernel(features, features_mask, codebook):
    B, T, H = features.shape
    N = codebook.shape[0]
    mask3 = features_mask.reshape(B, 1, T)
    idx3, sqm = pl.pallas_call(
        _vq_body,
        grid=(B,),
        in_specs=[
            pl.BlockSpec((1, T, H), lambda i: (i, 0, 0)),
            pl.BlockSpec((1, 1, T), lambda i: (i, 0, 0)),
            pl.BlockSpec((N, H), lambda i: (0, 0)),
        ],
        out_specs=[
            pl.BlockSpec((1, 1, T), lambda i: (i, 0, 0)),
            pl.BlockSpec((1, 1, T), lambda i: (i, 0, 0)),
        ],
        out_shape=[
            jax.ShapeDtypeStruct((B, 1, T), jnp.int32),
            jax.ShapeDtypeStruct((B, 1, T), jnp.float32),
        ],
    )(features, mask3, codebook)
    idx_flat = idx3.reshape(B * T)
    q = _make_sc_gather(N, H, B * T, 96)(codebook, idx_flat).reshape(B, T, H)
    mask_sum = jnp.sum(features_mask, axis=1)
    loss = jnp.sum(sqm[:, 0, :], axis=1) / mask_sum
    return (q, loss, loss)
